# baseline (speedup 1.0000x reference)
import functools

import jax
import jax.numpy as jnp
from jax.experimental import pallas as pl
from jax.experimental.pallas import tpu as pltpu

T = 200
H = 256
D = 512

_DNT = (((1,), (1,)), ((), ()))


def _dot_t(x, w):
    return jax.lax.dot_general(x, w, _DNT, preferred_element_type=jnp.float32)


def _lstm_kernel(imgs_ref, cells_ref, emb_i_hbm, emb_c_ref, w_ih_hbm,
                 w_hh_hbm, b_ih_ref, b_hh_ref, fc_w_ref, fc_b_ref, out_ref,
                 z_ref, hs_ref, w_hh_t_ref, emb_i_v, w_ih_v, w_hh_v,
                 sem_e, sem_i, sem_h):
    cp_h = pltpu.make_async_copy(w_hh_hbm, w_hh_v, sem_h)
    cp_e = pltpu.make_async_copy(emb_i_hbm, emb_i_v, sem_e)
    cp_i = pltpu.make_async_copy(w_ih_hbm, w_ih_v, sem_i)
    cp_h.start()
    cp_e.start()
    cp_i.start()

    img_ids = imgs_ref[:, 15:16]
    cell_ids = cells_ref[:, 15:16]
    oh_img = (jax.lax.broadcasted_iota(jnp.int32, (T, 900), 1)
              == img_ids).astype(jnp.float32)
    oh_cell = (jax.lax.broadcasted_iota(jnp.int32, (T, 8), 1)
               == cell_ids).astype(jnp.float32)
    emb_c8 = jnp.pad(emb_c_ref[:], ((0, 3), (0, 0)))
    x_cell = jnp.dot(oh_cell, emb_c8, preferred_element_type=jnp.float32)

    cp_h.wait()
    w_hh_t_ref[:] = w_hh_v[:].astype(jnp.bfloat16).T

    cp_e.wait()
    x_img = jnp.dot(oh_img, emb_i_v[:], preferred_element_type=jnp.float32)

    cp_i.wait()
    z = (_dot_t(x_img, w_ih_v[:, 0:H])
         + _dot_t(x_cell, w_ih_v[:, H:D])
         + b_ih_ref[:] + b_hh_ref[:])
    z_ref[:] = z.astype(jnp.bfloat16)

    h = jnp.zeros((1, H), jnp.float32)
    c = jnp.zeros((1, H), jnp.float32)
    for t in range(T):
        g = z_ref[t:t + 1, :] + jnp.dot(
            h.astype(jnp.bfloat16), w_hh_t_ref[:],
            preferred_element_type=jnp.float32)
        i = 0.5 * jnp.tanh(0.5 * g[:, 0:H]) + 0.5
        f = 0.5 * jnp.tanh(0.5 * g[:, H:2 * H]) + 0.5
        gg = jnp.tanh(g[:, 2 * H:3 * H])
        o = 0.5 * jnp.tanh(0.5 * g[:, 3 * H:4 * H]) + 0.5
        c = f * c + i * gg
        h = o * jnp.tanh(c)
        hs_ref[t:t + 1, :] = h

    logits = _dot_t(hs_ref[:], fc_w_ref[:]) + fc_b_ref[:]
    out_ref[:] = jax.nn.sigmoid(logits)


@functools.partial(jax.jit, static_argnames=("interpret",))
def _run(imgs, cells, emb_indice, emb_cell, w_ih, w_hh_t, b_ih, b_hh, fc_w,
         fc_b, interpret=False):
    vmem = pl.BlockSpec(memory_space=pltpu.MemorySpace.VMEM)
    hbm = pl.BlockSpec(memory_space=pltpu.MemorySpace.HBM)
    return pl.pallas_call(
        _lstm_kernel,
        out_shape=jax.ShapeDtypeStruct((T, 2), jnp.float32),
        in_specs=[vmem, vmem, hbm, vmem, hbm, hbm, vmem, vmem, vmem, vmem],
        scratch_shapes=[
            pltpu.VMEM((T, 4 * H), jnp.bfloat16),
            pltpu.VMEM((T, H), jnp.float32),
            pltpu.VMEM((H, 4 * H), jnp.bfloat16),
            pltpu.VMEM((900, H), jnp.float32),
            pltpu.VMEM((4 * H, D), jnp.float32),
            pltpu.VMEM((4 * H, H), jnp.float32),
            pltpu.SemaphoreType.DMA,
            pltpu.SemaphoreType.DMA,
            pltpu.SemaphoreType.DMA,
        ],
        interpret=interpret,
    )(imgs, cells, emb_indice, emb_cell, w_ih, w_hh_t, b_ih, b_hh, fc_w,
      fc_b)


def kernel(cells, imgs, emb_cell, emb_indice, W_ih, W_hh, b_ih, b_hh, fc_w,
           fc_b):
    return _run(imgs.astype(jnp.int32), cells.astype(jnp.int32), emb_indice,
                emb_cell, W_ih, W_hh,
                b_ih.reshape(1, 4 * H), b_hh.reshape(1, 4 * H), fc_w,
                fc_b.reshape(1, 2))


# speedup vs baseline: 1.0302x; 1.0302x over previous
import functools

import jax
import jax.numpy as jnp
from jax.experimental import pallas as pl
from jax.experimental.pallas import tpu as pltpu

T = 200
H = 256
D = 512

_DNT = (((1,), (1,)), ((), ()))


def _dot_t(x, w):
    return jax.lax.dot_general(x, w, _DNT, preferred_element_type=jnp.float32)


def _lstm_kernel(imgs_ref, cells_ref, emb_i_ref, emb_c_ref, w_ih_ref,
                 w_hh_ref, b_ih_ref, b_hh_ref, fc_w_ref, fc_b_ref, out_ref,
                 z_ref, hs_ref, w_hh_t_ref):
    w_hh_t_ref[:] = w_hh_ref[:].astype(jnp.bfloat16).T

    img_ids = imgs_ref[:, 15:16]
    cell_ids = cells_ref[:, 15:16]
    oh_img = (jax.lax.broadcasted_iota(jnp.int32, (T, 900), 1)
              == img_ids).astype(jnp.float32)
    oh_cell = (jax.lax.broadcasted_iota(jnp.int32, (T, 8), 1)
               == cell_ids).astype(jnp.float32)
    x_img = jnp.dot(oh_img, emb_i_ref[:], preferred_element_type=jnp.float32)
    emb_c8 = jnp.pad(emb_c_ref[:], ((0, 3), (0, 0)))
    x_cell = jnp.dot(oh_cell, emb_c8, preferred_element_type=jnp.float32)

    z = (_dot_t(x_img, w_ih_ref[:, 0:H])
         + _dot_t(x_cell, w_ih_ref[:, H:D])
         + b_ih_ref[:] + b_hh_ref[:])
    z_ref[:] = z.astype(jnp.bfloat16)

    h = jnp.zeros((1, H), jnp.float32)
    c = jnp.zeros((1, H), jnp.float32)
    for t in range(T):
        g = z_ref[t:t + 1, :] + jnp.dot(
            h.astype(jnp.bfloat16), w_hh_t_ref[:],
            preferred_element_type=jnp.float32)
        i = 0.5 * jnp.tanh(0.5 * g[:, 0:H]) + 0.5
        f = 0.5 * jnp.tanh(0.5 * g[:, H:2 * H]) + 0.5
        gg = jnp.tanh(g[:, 2 * H:3 * H])
        o = 0.5 * jnp.tanh(0.5 * g[:, 3 * H:4 * H]) + 0.5
        c = f * c + i * gg
        h = o * jnp.tanh(c)
        hs_ref[t:t + 1, :] = h

    logits = _dot_t(hs_ref[:], fc_w_ref[:]) + fc_b_ref[:]
    out_ref[:] = jax.nn.sigmoid(logits)


@functools.partial(jax.jit, static_argnames=("interpret",))
def _run(imgs, cells, emb_indice, emb_cell, w_ih, w_hh_t, b_ih, b_hh, fc_w,
         fc_b, interpret=False):
    return pl.pallas_call(
        _lstm_kernel,
        out_shape=jax.ShapeDtypeStruct((T, 2), jnp.float32),
        scratch_shapes=[
            pltpu.VMEM((T, 4 * H), jnp.bfloat16),
            pltpu.VMEM((T, H), jnp.float32),
            pltpu.VMEM((H, 4 * H), jnp.bfloat16),
        ],
        interpret=interpret,
    )(imgs, cells, emb_indice, emb_cell, w_ih, w_hh_t, b_ih, b_hh, fc_w,
      fc_b)


def kernel(cells, imgs, emb_cell, emb_indice, W_ih, W_hh, b_ih, b_hh, fc_w,
           fc_b):
    return _run(imgs.astype(jnp.int32), cells.astype(jnp.int32), emb_indice,
                emb_cell, W_ih, W_hh,
                b_ih.reshape(1, 4 * H), b_hh.reshape(1, 4 * H), fc_w,
                fc_b.reshape(1, 2))
